# baseline (speedup 1.0000x reference)
import functools

import jax
import jax.numpy as jnp
from jax import lax
from jax.experimental import pallas as pl
from jax.experimental.pallas import tpu as pltpu
from jax.experimental.pallas import tpu_sc as plsc

N = 10000
NP = 10240
DH = 64
HH = DH // 2
DUMMY = 10000
C = 128
GROUP = 8
PCHUNKS = 160
E_PAD = 16 * PCHUNKS * C
DCHUNKS = 80
STRIPE = NP // 16
DEG_W = 16
DEG_GROUP = 8


@functools.cache
def _mesh():
    return plsc.VectorSubcoreMesh(core_axis_name="c", subcore_axis_name="s",
                                  num_cores=2, num_subcores=16)


def _edge_sweep(u_sh, acc, src_v, dst_v, rows_v, gsem, ssem, gn=GROUP):
    def group(g, carry):
        @pl.when(g > 0)
        def _():
            for b in range(gn):
                pltpu.make_async_copy(rows_v.at[b], acc.at[dst_v.at[0]], ssem).wait()
        for b in range(gn):
            pltpu.async_copy(u_sh.at[src_v.at[g * gn + b]], rows_v.at[b], gsem)
        for b in range(gn):
            pltpu.make_async_copy(u_sh.at[src_v.at[0]], rows_v.at[b], gsem).wait()
        for b in range(gn):
            pltpu.async_copy(rows_v.at[b], acc.at[dst_v.at[g * gn + b]], ssem,
                             add=True)
        return carry

    lax.fori_loop(0, PCHUNKS // gn, group, 0)
    for b in range(gn):
        pltpu.make_async_copy(rows_v.at[b], acc.at[dst_v.at[0]], ssem).wait()


def _prop_body(u_hbm, srcs_hbm, dsts_hbm, out_hbm,
               src_v, dst_v, rows_v, acc, u_sh, gsem, ssem):
    c = lax.axis_index("c")
    s = lax.axis_index("s")
    pltpu.sync_copy(u_hbm.at[c, pl.ds(s * STRIPE, STRIPE)],
                    u_sh.at[pl.ds(s * STRIPE, STRIPE)])
    pltpu.sync_copy(u_hbm.at[c, pl.ds(s * STRIPE, STRIPE)],
                    acc.at[pl.ds(s * STRIPE, STRIPE)])
    pltpu.sync_copy(srcs_hbm.at[pl.ds(s * PCHUNKS, PCHUNKS)], src_v)
    pltpu.sync_copy(dsts_hbm.at[pl.ds(s * PCHUNKS, PCHUNKS)], dst_v)
    plsc.subcore_barrier()
    _edge_sweep(u_sh, acc, src_v, dst_v, rows_v, gsem, ssem)
    plsc.subcore_barrier()
    pltpu.sync_copy(acc.at[pl.ds(s * STRIPE, STRIPE)],
                    out_hbm.at[c, pl.ds(s * STRIPE, STRIPE)])


@functools.cache
def _prop_call():
    return pl.kernel(
        _prop_body,
        out_type=jax.ShapeDtypeStruct((2, NP, HH), jnp.float32),
        mesh=_mesh(),
        scratch_types=[
            pltpu.VMEM((PCHUNKS, C), jnp.int32),
            pltpu.VMEM((PCHUNKS, C), jnp.int32),
            pltpu.VMEM((GROUP, C, HH), jnp.float32),
            pltpu.VMEM_SHARED((NP, HH), jnp.float32),
            pltpu.VMEM_SHARED((NP, HH), jnp.float32),
            pltpu.SemaphoreType.DMA,
            pltpu.SemaphoreType.DMA,
        ],
        compiler_params=pltpu.CompilerParams(use_tc_tiling_on_sc=False),
    )


def _prop23_body(ulo_hbm, uhi_hbm, srcs_hbm, dsts_hbm, dis_hbm, a_hbm, b_hbm,
                 olo_hbm, ohi_hbm, src_v, dst_v, rows_v, sbuf, dis_v, a_v,
                 b_v, acc, u_sh, gsem, ssem):
    c = lax.axis_index("c")
    s = lax.axis_index("s")

    @pl.when(c == 0)
    def _():
        pltpu.sync_copy(ulo_hbm.at[pl.ds(s * STRIPE, STRIPE)],
                        u_sh.at[pl.ds(s * STRIPE, STRIPE)])
        pltpu.sync_copy(ulo_hbm.at[pl.ds(s * STRIPE, STRIPE)],
                        acc.at[pl.ds(s * STRIPE, STRIPE)])

    @pl.when(c == 1)
    def _():
        pltpu.sync_copy(uhi_hbm.at[pl.ds(s * STRIPE, STRIPE)],
                        u_sh.at[pl.ds(s * STRIPE, STRIPE)])
        pltpu.sync_copy(uhi_hbm.at[pl.ds(s * STRIPE, STRIPE)],
                        acc.at[pl.ds(s * STRIPE, STRIPE)])

    pltpu.sync_copy(srcs_hbm.at[pl.ds(s * PCHUNKS, PCHUNKS)], src_v)
    pltpu.sync_copy(dsts_hbm.at[pl.ds(s * PCHUNKS, PCHUNKS)], dst_v)
    pltpu.sync_copy(dis_hbm.at[pl.ds(s * STRIPE, STRIPE)], dis_v)
    pltpu.sync_copy(a_hbm.at[c], a_v)
    pltpu.sync_copy(b_hbm.at[c], b_v)
    plsc.subcore_barrier()
    _edge_sweep(u_sh, acc, src_v, dst_v, rows_v, gsem, ssem, gn=4)
    plsc.subcore_barrier()
    pltpu.sync_copy(acc.at[pl.ds(s * STRIPE, STRIPE)], sbuf)
    a0 = a_v[pl.ds(0, 16)]
    a1 = a_v[pl.ds(16, 16)]
    b0 = b_v[pl.ds(0, 16)]
    b1 = b_v[pl.ds(16, 16)]

    def row(j, carry):
        d = plsc.load_gather(dis_v, [jnp.full((16,), j, jnp.int32)])
        r0 = sbuf[j, pl.ds(0, 16)]
        r1 = sbuf[j, pl.ds(16, 16)]
        sbuf[j, pl.ds(0, 16)] = d * jnp.maximum(d * r0 * a0 + b0, 0.0)
        sbuf[j, pl.ds(16, 16)] = d * jnp.maximum(d * r1 * a1 + b1, 0.0)
        return carry

    lax.fori_loop(0, STRIPE, row, 0)
    pltpu.sync_copy(sbuf, u_sh.at[pl.ds(s * STRIPE, STRIPE)])
    pltpu.sync_copy(sbuf, acc.at[pl.ds(s * STRIPE, STRIPE)])
    plsc.subcore_barrier()
    _edge_sweep(u_sh, acc, src_v, dst_v, rows_v, gsem, ssem, gn=4)
    plsc.subcore_barrier()

    @pl.when(c == 0)
    def _():
        pltpu.sync_copy(acc.at[pl.ds(s * STRIPE, STRIPE)],
                        olo_hbm.at[pl.ds(s * STRIPE, STRIPE)])

    @pl.when(c == 1)
    def _():
        pltpu.sync_copy(acc.at[pl.ds(s * STRIPE, STRIPE)],
                        ohi_hbm.at[pl.ds(s * STRIPE, STRIPE)])


@functools.cache
def _prop23_call():
    return pl.kernel(
        _prop23_body,
        out_type=[jax.ShapeDtypeStruct((NP, HH), jnp.float32),
                  jax.ShapeDtypeStruct((NP, HH), jnp.float32)],
        mesh=_mesh(),
        scratch_types=[
            pltpu.VMEM((PCHUNKS, C), jnp.int32),
            pltpu.VMEM((PCHUNKS, C), jnp.int32),
            pltpu.VMEM((4, C, HH), jnp.float32),
            pltpu.VMEM((STRIPE, HH), jnp.float32),
            pltpu.VMEM((STRIPE,), jnp.float32),
            pltpu.VMEM((HH,), jnp.float32),
            pltpu.VMEM((HH,), jnp.float32),
            pltpu.VMEM_SHARED((NP, HH), jnp.float32),
            pltpu.VMEM_SHARED((NP, HH), jnp.float32),
            pltpu.SemaphoreType.DMA,
            pltpu.SemaphoreType.DMA,
        ],
        compiler_params=pltpu.CompilerParams(use_tc_tiling_on_sc=False,
                                             needs_layout_passes=False),
    )


def _deg_body(dsts_hbm, ones_hbm, zeros_hbm, out_hbm,
              dst_v, ones_v, acc, ssem):
    c = lax.axis_index("c")
    s = lax.axis_index("s")
    w = s * 2 + c
    pltpu.sync_copy(zeros_hbm.at[pl.ds(s * STRIPE, STRIPE)],
                    acc.at[pl.ds(s * STRIPE, STRIPE)])
    pltpu.sync_copy(ones_hbm, ones_v)
    pltpu.sync_copy(dsts_hbm.at[pl.ds(w * DCHUNKS, DCHUNKS)], dst_v)
    plsc.subcore_barrier()

    def group(g, carry):
        @pl.when(g > 0)
        def _():
            for b in range(DEG_GROUP):
                pltpu.make_async_copy(ones_v, acc.at[dst_v.at[0]], ssem).wait()
        for b in range(DEG_GROUP):
            pltpu.async_copy(ones_v, acc.at[dst_v.at[g * DEG_GROUP + b]], ssem,
                             add=True)
        return carry

    lax.fori_loop(0, DCHUNKS // DEG_GROUP, group, 0)
    for b in range(DEG_GROUP):
        pltpu.make_async_copy(ones_v, acc.at[dst_v.at[0]], ssem).wait()
    plsc.subcore_barrier()
    pltpu.sync_copy(acc.at[pl.ds(s * STRIPE, STRIPE)],
                    out_hbm.at[c, pl.ds(s * STRIPE, STRIPE)])


@functools.cache
def _deg_call():
    return pl.kernel(
        _deg_body,
        out_type=jax.ShapeDtypeStruct((2, NP, DEG_W), jnp.float32),
        mesh=_mesh(),
        scratch_types=[
            pltpu.VMEM((DCHUNKS, C), jnp.int32),
            pltpu.VMEM((C, DEG_W), jnp.float32),
            pltpu.VMEM_SHARED((NP, DEG_W), jnp.float32),
            pltpu.SemaphoreType.DMA,
        ],
        compiler_params=pltpu.CompilerParams(use_tc_tiling_on_sc=False),
    )


_RB = 1024
_GRID = NP // _RB

_halves = pl.BlockSpec((2, _RB, HH), lambda i: (0, i, 0))


def _split(u, out_ref):
    out_ref[0] = u[:, :HH]
    out_ref[1] = u[:, HH:]


def _tc1_body(x_ref, w1_ref, d0_ref, d1_ref, u1_ref, dis_ref):
    cnt = d0_ref[:, 0:1] + d1_ref[:, 0:1]
    dis = lax.rsqrt(cnt + 1.0)
    dis_ref[...] = dis
    h = jnp.dot(x_ref[...], w1_ref[...], preferred_element_type=jnp.float32)
    _split(dis * h, u1_ref)


def _tc1(x, w1, d0, d1):
    return pl.pallas_call(
        _tc1_body,
        grid=(_GRID,),
        in_specs=[
            pl.BlockSpec((_RB, 128), lambda i: (i, 0)),
            pl.BlockSpec((128, DH), lambda i: (0, 0)),
            pl.BlockSpec((_RB, DEG_W), lambda i: (i, 0)),
            pl.BlockSpec((_RB, DEG_W), lambda i: (i, 0)),
        ],
        out_specs=[
            _halves,
            pl.BlockSpec((_RB, 1), lambda i: (i, 0)),
        ],
        out_shape=[
            jax.ShapeDtypeStruct((2, NP, HH), jnp.float32),
            jax.ShapeDtypeStruct((NP, 1), jnp.float32),
        ],
    )(x, w1, d0, d1)


def _tc2_body(s_ref, dis_ref, b_ref, g_ref, be_ref, rm_ref,
              rv_ref, w_ref, out_ref):
    dis = dis_ref[...]
    su = jnp.concatenate([s_ref[0], s_ref[1]], axis=1)
    t = dis * su + b_ref[...]
    t = (t - rm_ref[...]) * lax.rsqrt(rv_ref[...] + 1e-5) * g_ref[...] + be_ref[...]
    t = jnp.maximum(t, 0.0)
    _split(dis * jnp.dot(t, w_ref[...], preferred_element_type=jnp.float32),
           out_ref)


def _tc2(s, dis, b, g, be, rm, rv, w):
    vec = pl.BlockSpec((1, DH), lambda i: (0, 0))
    return pl.pallas_call(
        _tc2_body,
        grid=(_GRID,),
        in_specs=[_halves, pl.BlockSpec((_RB, 1), lambda i: (i, 0)),
                  vec, vec, vec, vec, vec,
                  pl.BlockSpec((DH, DH), lambda i: (0, 0))],
        out_specs=_halves,
        out_shape=jax.ShapeDtypeStruct((2, NP, HH), jnp.float32),
    )(s, dis, b, g, be, rm, rv, w)


def _tc3_body(s_ref, dis_ref, b_ref, g_ref, be_ref, rm_ref, rv_ref, out_ref):
    dis = dis_ref[...]
    su = jnp.concatenate([s_ref[0], s_ref[1]], axis=1)
    t = dis * su + b_ref[...]
    t = (t - rm_ref[...]) * lax.rsqrt(rv_ref[...] + 1e-5) * g_ref[...] + be_ref[...]
    t = jnp.maximum(t, 0.0)
    _split(dis * t, out_ref)


def _tc3(s, dis, b, g, be, rm, rv):
    vec = pl.BlockSpec((1, DH), lambda i: (0, 0))
    return pl.pallas_call(
        _tc3_body,
        grid=(_GRID,),
        in_specs=[_halves, pl.BlockSpec((_RB, 1), lambda i: (i, 0)),
                  vec, vec, vec, vec, vec],
        out_specs=_halves,
        out_shape=jax.ShapeDtypeStruct((2, NP, HH), jnp.float32),
    )(s, dis, b, g, be, rm, rv)


def _tc4_body(s_ref, dis_ref, w_ref, b_ref, out_ref):
    p = dis_ref[...] * jnp.concatenate([s_ref[0], s_ref[1]], axis=1)
    out_ref[...] = jnp.dot(p, w_ref[...],
                           preferred_element_type=jnp.float32) + b_ref[...]


def _tc4(s, dis, w, b):
    return pl.pallas_call(
        _tc4_body,
        grid=(_GRID,),
        in_specs=[_halves, pl.BlockSpec((_RB, 1), lambda i: (i, 0)),
                  pl.BlockSpec((DH, 128), lambda i: (0, 0)),
                  pl.BlockSpec((1, 128), lambda i: (0, 0))],
        out_specs=pl.BlockSpec((_RB, 128), lambda i: (i, 0)),
        out_shape=jax.ShapeDtypeStruct((NP, 128), jnp.float32),
    )(s, dis, w, b)


def kernel(x, edge_index, W1, b1, g1, be1, rm1, rv1, W2, b2, g2, be2, rm2,
           rv2, W3, b3):
    src = edge_index[0].astype(jnp.int32)
    dst = edge_index[1].astype(jnp.int32)
    pad = E_PAD - src.shape[0]
    srcs = jnp.concatenate([src, jnp.zeros((pad,), jnp.int32)]).reshape(-1, C)
    dsts = jnp.concatenate([dst, jnp.full((pad,), DUMMY, jnp.int32)]).reshape(-1, C)
    ones = jnp.ones((C, DEG_W), jnp.float32)
    zeros = jnp.zeros((NP, DEG_W), jnp.float32)
    xp = jnp.pad(x, ((0, NP - x.shape[0]), (0, 0)))

    degp = _deg_call()(dsts, ones, zeros)
    u1, dis = _tc1(xp, W1, degp[0], degp[1])
    s1 = _prop_call()(u1, srcs, dsts)
    u2 = _tc2(s1, dis, b1.reshape(1, DH), g1.reshape(1, DH),
              be1.reshape(1, DH), rm1.reshape(1, DH), rv1.reshape(1, DH), W2)
    a2 = g2 * lax.rsqrt(rv2 + 1e-5)
    b2f = (b2 - rm2) * a2 + be2
    s3lo, s3hi = _prop23_call()(u2[0], u2[1], srcs, dsts, dis.reshape(NP),
                                a2.reshape(2, HH), b2f.reshape(2, HH))
    s3 = jnp.stack([s3lo, s3hi])
    w3p = jnp.pad(W3, ((0, 0), (0, 128 - W3.shape[1])))
    b3p = jnp.pad(b3, (0, 128 - b3.shape[0])).reshape(1, 128)
    outp = _tc4(s3, dis, w3p, b3p)
    return outp[:N, :b3.shape[0]]


# speedup vs baseline: 1.0727x; 1.0727x over previous
import functools

import jax
import jax.numpy as jnp
from jax import lax
from jax.experimental import pallas as pl
from jax.experimental.pallas import tpu as pltpu
from jax.experimental.pallas import tpu_sc as plsc

N = 10000
NP = 10240
DH = 64
HH = DH // 2
DUMMY = 10000
C = 128
GROUP = 8
PCHUNKS = 160
E_PAD = 16 * PCHUNKS * C
DCHUNKS = 80
STRIPE = NP // 16
DEG_W = 16
DEG_GROUP = 8
SLAB = 128


@functools.cache
def _mesh():
    return plsc.VectorSubcoreMesh(core_axis_name="c", subcore_axis_name="s",
                                  num_cores=2, num_subcores=16)


def _edge_sweep(u_sh, acc, src_v, dst_v, rows_v, gsem, ssem, gn=GROUP):
    def group(g, carry):
        @pl.when(g > 0)
        def _():
            for b in range(gn):
                pltpu.make_async_copy(rows_v.at[b], acc.at[dst_v.at[0]], ssem).wait()
        for b in range(gn):
            pltpu.async_copy(u_sh.at[src_v.at[g * gn + b]], rows_v.at[b], gsem)
        for b in range(gn):
            pltpu.make_async_copy(u_sh.at[src_v.at[0]], rows_v.at[b], gsem).wait()
        for b in range(gn):
            pltpu.async_copy(rows_v.at[b], acc.at[dst_v.at[g * gn + b]], ssem,
                             add=True)
        return carry

    lax.fori_loop(0, PCHUNKS // gn, group, 0)
    for b in range(gn):
        pltpu.make_async_copy(rows_v.at[b], acc.at[dst_v.at[0]], ssem).wait()


def _prop_body(u_hbm, srcs_hbm, dsts_hbm, out_hbm,
               src_v, dst_v, rows_v, acc, u_sh, gsem, ssem):
    c = lax.axis_index("c")
    s = lax.axis_index("s")
    pltpu.sync_copy(u_hbm.at[c, pl.ds(s * STRIPE, STRIPE)],
                    u_sh.at[pl.ds(s * STRIPE, STRIPE)])
    pltpu.sync_copy(u_hbm.at[c, pl.ds(s * STRIPE, STRIPE)],
                    acc.at[pl.ds(s * STRIPE, STRIPE)])
    pltpu.sync_copy(srcs_hbm.at[pl.ds(s * PCHUNKS, PCHUNKS)], src_v)
    pltpu.sync_copy(dsts_hbm.at[pl.ds(s * PCHUNKS, PCHUNKS)], dst_v)
    plsc.subcore_barrier()
    _edge_sweep(u_sh, acc, src_v, dst_v, rows_v, gsem, ssem)
    plsc.subcore_barrier()
    pltpu.sync_copy(acc.at[pl.ds(s * STRIPE, STRIPE)],
                    out_hbm.at[c, pl.ds(s * STRIPE, STRIPE)])


@functools.cache
def _prop_call():
    return pl.kernel(
        _prop_body,
        out_type=jax.ShapeDtypeStruct((2, NP, HH), jnp.float32),
        mesh=_mesh(),
        scratch_types=[
            pltpu.VMEM((PCHUNKS, C), jnp.int32),
            pltpu.VMEM((PCHUNKS, C), jnp.int32),
            pltpu.VMEM((GROUP, C, HH), jnp.float32),
            pltpu.VMEM_SHARED((NP, HH), jnp.float32),
            pltpu.VMEM_SHARED((NP, HH), jnp.float32),
            pltpu.SemaphoreType.DMA,
            pltpu.SemaphoreType.DMA,
        ],
        compiler_params=pltpu.CompilerParams(use_tc_tiling_on_sc=False),
    )


def _prop23_body(ulo_hbm, uhi_hbm, srcs_hbm, dsts_hbm, dis_hbm, a_hbm, b_hbm,
                 olo_hbm, ohi_hbm, src_v, dst_v, rows_v, sbuf, dis_v, a_v,
                 b_v, acc, u_sh, gsem, ssem):
    c = lax.axis_index("c")
    s = lax.axis_index("s")

    @pl.when(c == 0)
    def _():
        pltpu.sync_copy(ulo_hbm.at[pl.ds(s * STRIPE, STRIPE)],
                        u_sh.at[pl.ds(s * STRIPE, STRIPE)])
        pltpu.sync_copy(ulo_hbm.at[pl.ds(s * STRIPE, STRIPE)],
                        acc.at[pl.ds(s * STRIPE, STRIPE)])

    @pl.when(c == 1)
    def _():
        pltpu.sync_copy(uhi_hbm.at[pl.ds(s * STRIPE, STRIPE)],
                        u_sh.at[pl.ds(s * STRIPE, STRIPE)])
        pltpu.sync_copy(uhi_hbm.at[pl.ds(s * STRIPE, STRIPE)],
                        acc.at[pl.ds(s * STRIPE, STRIPE)])

    pltpu.sync_copy(srcs_hbm.at[pl.ds(s * PCHUNKS, PCHUNKS)], src_v)
    pltpu.sync_copy(dsts_hbm.at[pl.ds(s * PCHUNKS, PCHUNKS)], dst_v)
    pltpu.sync_copy(dis_hbm.at[pl.ds(s * STRIPE, STRIPE)], dis_v)
    pltpu.sync_copy(a_hbm.at[c], a_v)
    pltpu.sync_copy(b_hbm.at[c], b_v)
    plsc.subcore_barrier()
    _edge_sweep(u_sh, acc, src_v, dst_v, rows_v, gsem, ssem)
    plsc.subcore_barrier()
    a0 = a_v[pl.ds(0, 16)]
    a1 = a_v[pl.ds(16, 16)]
    b0 = b_v[pl.ds(0, 16)]
    b1 = b_v[pl.ds(16, 16)]

    def slab(t, carry):
        base = s * STRIPE + t * SLAB
        pltpu.sync_copy(acc.at[pl.ds(base, SLAB)], sbuf)

        def row(j, carry2):
            d = plsc.load_gather(dis_v,
                                 [jnp.full((16,), t * SLAB + j, jnp.int32)])
            r0 = sbuf[j, pl.ds(0, 16)]
            r1 = sbuf[j, pl.ds(16, 16)]
            sbuf[j, pl.ds(0, 16)] = d * jnp.maximum(d * r0 * a0 + b0, 0.0)
            sbuf[j, pl.ds(16, 16)] = d * jnp.maximum(d * r1 * a1 + b1, 0.0)
            return carry2

        lax.fori_loop(0, SLAB, row, 0, unroll=4)
        pltpu.sync_copy(sbuf, u_sh.at[pl.ds(base, SLAB)])
        pltpu.sync_copy(sbuf, acc.at[pl.ds(base, SLAB)])
        return carry

    lax.fori_loop(0, STRIPE // SLAB, slab, 0)
    plsc.subcore_barrier()
    _edge_sweep(u_sh, acc, src_v, dst_v, rows_v, gsem, ssem)
    plsc.subcore_barrier()

    @pl.when(c == 0)
    def _():
        pltpu.sync_copy(acc.at[pl.ds(s * STRIPE, STRIPE)],
                        olo_hbm.at[pl.ds(s * STRIPE, STRIPE)])

    @pl.when(c == 1)
    def _():
        pltpu.sync_copy(acc.at[pl.ds(s * STRIPE, STRIPE)],
                        ohi_hbm.at[pl.ds(s * STRIPE, STRIPE)])


@functools.cache
def _prop23_call():
    return pl.kernel(
        _prop23_body,
        out_type=[jax.ShapeDtypeStruct((NP, HH), jnp.float32),
                  jax.ShapeDtypeStruct((NP, HH), jnp.float32)],
        mesh=_mesh(),
        scratch_types=[
            pltpu.VMEM((PCHUNKS, C), jnp.int32),
            pltpu.VMEM((PCHUNKS, C), jnp.int32),
            pltpu.VMEM((GROUP, C, HH), jnp.float32),
            pltpu.VMEM((SLAB, HH), jnp.float32),
            pltpu.VMEM((STRIPE,), jnp.float32),
            pltpu.VMEM((HH,), jnp.float32),
            pltpu.VMEM((HH,), jnp.float32),
            pltpu.VMEM_SHARED((NP, HH), jnp.float32),
            pltpu.VMEM_SHARED((NP, HH), jnp.float32),
            pltpu.SemaphoreType.DMA,
            pltpu.SemaphoreType.DMA,
        ],
        compiler_params=pltpu.CompilerParams(use_tc_tiling_on_sc=False,
                                             needs_layout_passes=False),
    )


def _deg_body(dsts_hbm, ones_hbm, zeros_hbm, out_hbm,
              dst_v, ones_v, acc, ssem):
    c = lax.axis_index("c")
    s = lax.axis_index("s")
    w = s * 2 + c
    pltpu.sync_copy(zeros_hbm.at[pl.ds(s * STRIPE, STRIPE)],
                    acc.at[pl.ds(s * STRIPE, STRIPE)])
    pltpu.sync_copy(ones_hbm, ones_v)
    pltpu.sync_copy(dsts_hbm.at[pl.ds(w * DCHUNKS, DCHUNKS)], dst_v)
    plsc.subcore_barrier()

    def group(g, carry):
        @pl.when(g > 0)
        def _():
            for b in range(DEG_GROUP):
                pltpu.make_async_copy(ones_v, acc.at[dst_v.at[0]], ssem).wait()
        for b in range(DEG_GROUP):
            pltpu.async_copy(ones_v, acc.at[dst_v.at[g * DEG_GROUP + b]], ssem,
                             add=True)
        return carry

    lax.fori_loop(0, DCHUNKS // DEG_GROUP, group, 0)
    for b in range(DEG_GROUP):
        pltpu.make_async_copy(ones_v, acc.at[dst_v.at[0]], ssem).wait()
    plsc.subcore_barrier()
    pltpu.sync_copy(acc.at[pl.ds(s * STRIPE, STRIPE)],
                    out_hbm.at[c, pl.ds(s * STRIPE, STRIPE)])


@functools.cache
def _deg_call():
    return pl.kernel(
        _deg_body,
        out_type=jax.ShapeDtypeStruct((2, NP, DEG_W), jnp.float32),
        mesh=_mesh(),
        scratch_types=[
            pltpu.VMEM((DCHUNKS, C), jnp.int32),
            pltpu.VMEM((C, DEG_W), jnp.float32),
            pltpu.VMEM_SHARED((NP, DEG_W), jnp.float32),
            pltpu.SemaphoreType.DMA,
        ],
        compiler_params=pltpu.CompilerParams(use_tc_tiling_on_sc=False),
    )


_RB = 1024
_GRID = NP // _RB

_halves = pl.BlockSpec((2, _RB, HH), lambda i: (0, i, 0))


def _split(u, out_ref):
    out_ref[0] = u[:, :HH]
    out_ref[1] = u[:, HH:]


def _tc1_body(x_ref, w1_ref, d0_ref, d1_ref, u1_ref, dis_ref):
    cnt = d0_ref[:, 0:1] + d1_ref[:, 0:1]
    dis = lax.rsqrt(cnt + 1.0)
    dis_ref[...] = dis
    h = jnp.dot(x_ref[...], w1_ref[...], preferred_element_type=jnp.float32)
    _split(dis * h, u1_ref)


def _tc1(x, w1, d0, d1):
    return pl.pallas_call(
        _tc1_body,
        grid=(_GRID,),
        in_specs=[
            pl.BlockSpec((_RB, 128), lambda i: (i, 0)),
            pl.BlockSpec((128, DH), lambda i: (0, 0)),
            pl.BlockSpec((_RB, DEG_W), lambda i: (i, 0)),
            pl.BlockSpec((_RB, DEG_W), lambda i: (i, 0)),
        ],
        out_specs=[
            _halves,
            pl.BlockSpec((_RB, 1), lambda i: (i, 0)),
        ],
        out_shape=[
            jax.ShapeDtypeStruct((2, NP, HH), jnp.float32),
            jax.ShapeDtypeStruct((NP, 1), jnp.float32),
        ],
    )(x, w1, d0, d1)


def _tc2_body(s_ref, dis_ref, b_ref, g_ref, be_ref, rm_ref,
              rv_ref, w_ref, out_ref):
    dis = dis_ref[...]
    su = jnp.concatenate([s_ref[0], s_ref[1]], axis=1)
    t = dis * su + b_ref[...]
    t = (t - rm_ref[...]) * lax.rsqrt(rv_ref[...] + 1e-5) * g_ref[...] + be_ref[...]
    t = jnp.maximum(t, 0.0)
    _split(dis * jnp.dot(t, w_ref[...], preferred_element_type=jnp.float32),
           out_ref)


def _tc2(s, dis, b, g, be, rm, rv, w):
    vec = pl.BlockSpec((1, DH), lambda i: (0, 0))
    return pl.pallas_call(
        _tc2_body,
        grid=(_GRID,),
        in_specs=[_halves, pl.BlockSpec((_RB, 1), lambda i: (i, 0)),
                  vec, vec, vec, vec, vec,
                  pl.BlockSpec((DH, DH), lambda i: (0, 0))],
        out_specs=_halves,
        out_shape=jax.ShapeDtypeStruct((2, NP, HH), jnp.float32),
    )(s, dis, b, g, be, rm, rv, w)


def _tc3_body(s_ref, dis_ref, b_ref, g_ref, be_ref, rm_ref, rv_ref, out_ref):
    dis = dis_ref[...]
    su = jnp.concatenate([s_ref[0], s_ref[1]], axis=1)
    t = dis * su + b_ref[...]
    t = (t - rm_ref[...]) * lax.rsqrt(rv_ref[...] + 1e-5) * g_ref[...] + be_ref[...]
    t = jnp.maximum(t, 0.0)
    _split(dis * t, out_ref)


def _tc3(s, dis, b, g, be, rm, rv):
    vec = pl.BlockSpec((1, DH), lambda i: (0, 0))
    return pl.pallas_call(
        _tc3_body,
        grid=(_GRID,),
        in_specs=[_halves, pl.BlockSpec((_RB, 1), lambda i: (i, 0)),
                  vec, vec, vec, vec, vec],
        out_specs=_halves,
        out_shape=jax.ShapeDtypeStruct((2, NP, HH), jnp.float32),
    )(s, dis, b, g, be, rm, rv)


def _tc4_body(s_ref, dis_ref, w_ref, b_ref, out_ref):
    p = dis_ref[...] * jnp.concatenate([s_ref[0], s_ref[1]], axis=1)
    out_ref[...] = jnp.dot(p, w_ref[...],
                           preferred_element_type=jnp.float32) + b_ref[...]


def _tc4(s, dis, w, b):
    return pl.pallas_call(
        _tc4_body,
        grid=(_GRID,),
        in_specs=[_halves, pl.BlockSpec((_RB, 1), lambda i: (i, 0)),
                  pl.BlockSpec((DH, 128), lambda i: (0, 0)),
                  pl.BlockSpec((1, 128), lambda i: (0, 0))],
        out_specs=pl.BlockSpec((_RB, 128), lambda i: (i, 0)),
        out_shape=jax.ShapeDtypeStruct((NP, 128), jnp.float32),
    )(s, dis, w, b)


def kernel(x, edge_index, W1, b1, g1, be1, rm1, rv1, W2, b2, g2, be2, rm2,
           rv2, W3, b3):
    src = edge_index[0].astype(jnp.int32)
    dst = edge_index[1].astype(jnp.int32)
    pad = E_PAD - src.shape[0]
    srcs = jnp.concatenate([src, jnp.zeros((pad,), jnp.int32)]).reshape(-1, C)
    dsts = jnp.concatenate([dst, jnp.full((pad,), DUMMY, jnp.int32)]).reshape(-1, C)
    ones = jnp.ones((C, DEG_W), jnp.float32)
    zeros = jnp.zeros((NP, DEG_W), jnp.float32)
    xp = jnp.pad(x, ((0, NP - x.shape[0]), (0, 0)))

    degp = _deg_call()(dsts, ones, zeros)
    u1, dis = _tc1(xp, W1, degp[0], degp[1])
    s1 = _prop_call()(u1, srcs, dsts)
    u2 = _tc2(s1, dis, b1.reshape(1, DH), g1.reshape(1, DH),
              be1.reshape(1, DH), rm1.reshape(1, DH), rv1.reshape(1, DH), W2)
    a2 = g2 * lax.rsqrt(rv2 + 1e-5)
    b2f = (b2 - rm2) * a2 + be2
    s3lo, s3hi = _prop23_call()(u2[0], u2[1], srcs, dsts, dis.reshape(NP),
                                a2.reshape(2, HH), b2f.reshape(2, HH))
    s3 = jnp.stack([s3lo, s3hi])
    w3p = jnp.pad(W3, ((0, 0), (0, 128 - W3.shape[1])))
    b3p = jnp.pad(b3, (0, 128 - b3.shape[0])).reshape(1, 128)
    outp = _tc4(s3, dis, w3p, b3p)
    return outp[:N, :b3.shape[0]]


# speedup vs baseline: 1.0787x; 1.0056x over previous
import functools

import jax
import jax.numpy as jnp
from jax import lax
from jax.experimental import pallas as pl
from jax.experimental.pallas import tpu as pltpu
from jax.experimental.pallas import tpu_sc as plsc

N = 10000
NP = 10240
DH = 64
HH = DH // 2
DUMMY = 10000
C = 128
GROUP = 8
PCHUNKS = 160
E_PAD = 16 * PCHUNKS * C
DCHUNKS = 80
STRIPE = NP // 16
DEG_W = 16
DEG_GROUP = 8
SLAB = 128


@functools.cache
def _mesh():
    return plsc.VectorSubcoreMesh(core_axis_name="c", subcore_axis_name="s",
                                  num_cores=2, num_subcores=16)


def _edge_sweep(u_sh, acc, src_v, dst_v, rows_v, gsem, ssem, gn=GROUP):
    def group(g, carry):
        @pl.when(g > 0)
        def _():
            for b in range(gn):
                pltpu.make_async_copy(rows_v.at[b], acc.at[dst_v.at[0]], ssem).wait()
        for b in range(gn):
            pltpu.async_copy(u_sh.at[src_v.at[g * gn + b]], rows_v.at[b], gsem)
        for b in range(gn):
            pltpu.make_async_copy(u_sh.at[src_v.at[0]], rows_v.at[b], gsem).wait()
        for b in range(gn):
            pltpu.async_copy(rows_v.at[b], acc.at[dst_v.at[g * gn + b]], ssem,
                             add=True)
        return carry

    lax.fori_loop(0, PCHUNKS // gn, group, 0)
    for b in range(gn):
        pltpu.make_async_copy(rows_v.at[b], acc.at[dst_v.at[0]], ssem).wait()


def _prop_body(u_hbm, srcs_hbm, dsts_hbm, out_hbm,
               src_v, dst_v, rows_v, acc, u_sh, gsem, ssem):
    c = lax.axis_index("c")
    s = lax.axis_index("s")
    pltpu.sync_copy(u_hbm.at[c, pl.ds(s * STRIPE, STRIPE)],
                    u_sh.at[pl.ds(s * STRIPE, STRIPE)])
    pltpu.sync_copy(u_hbm.at[c, pl.ds(s * STRIPE, STRIPE)],
                    acc.at[pl.ds(s * STRIPE, STRIPE)])
    pltpu.sync_copy(srcs_hbm.at[pl.ds(s * PCHUNKS, PCHUNKS)], src_v)
    pltpu.sync_copy(dsts_hbm.at[pl.ds(s * PCHUNKS, PCHUNKS)], dst_v)
    plsc.subcore_barrier()
    _edge_sweep(u_sh, acc, src_v, dst_v, rows_v, gsem, ssem)
    plsc.subcore_barrier()
    pltpu.sync_copy(acc.at[pl.ds(s * STRIPE, STRIPE)],
                    out_hbm.at[c, pl.ds(s * STRIPE, STRIPE)])


@functools.cache
def _prop_call():
    return pl.kernel(
        _prop_body,
        out_type=jax.ShapeDtypeStruct((2, NP, HH), jnp.float32),
        mesh=_mesh(),
        scratch_types=[
            pltpu.VMEM((PCHUNKS, C), jnp.int32),
            pltpu.VMEM((PCHUNKS, C), jnp.int32),
            pltpu.VMEM((GROUP, C, HH), jnp.float32),
            pltpu.VMEM_SHARED((NP, HH), jnp.float32),
            pltpu.VMEM_SHARED((NP, HH), jnp.float32),
            pltpu.SemaphoreType.DMA,
            pltpu.SemaphoreType.DMA,
        ],
        compiler_params=pltpu.CompilerParams(use_tc_tiling_on_sc=False),
    )


def _prop23_body(ulo_hbm, uhi_hbm, srcs_hbm, dsts_hbm, dis_hbm, a_hbm, b_hbm,
                 olo_hbm, ohi_hbm, src_v, dst_v, rows_v, sbuf, dis_v, a_v,
                 b_v, acc, u_sh, gsem, ssem):
    c = lax.axis_index("c")
    s = lax.axis_index("s")

    @pl.when(c == 0)
    def _():
        pltpu.sync_copy(ulo_hbm.at[pl.ds(s * STRIPE, STRIPE)],
                        u_sh.at[pl.ds(s * STRIPE, STRIPE)])
        pltpu.sync_copy(ulo_hbm.at[pl.ds(s * STRIPE, STRIPE)],
                        acc.at[pl.ds(s * STRIPE, STRIPE)])

    @pl.when(c == 1)
    def _():
        pltpu.sync_copy(uhi_hbm.at[pl.ds(s * STRIPE, STRIPE)],
                        u_sh.at[pl.ds(s * STRIPE, STRIPE)])
        pltpu.sync_copy(uhi_hbm.at[pl.ds(s * STRIPE, STRIPE)],
                        acc.at[pl.ds(s * STRIPE, STRIPE)])

    pltpu.sync_copy(srcs_hbm.at[pl.ds(s * PCHUNKS, PCHUNKS)], src_v)
    pltpu.sync_copy(dsts_hbm.at[pl.ds(s * PCHUNKS, PCHUNKS)], dst_v)
    pltpu.sync_copy(dis_hbm.at[pl.ds(s * STRIPE, STRIPE)], dis_v)
    pltpu.sync_copy(a_hbm.at[c], a_v)
    pltpu.sync_copy(b_hbm.at[c], b_v)
    plsc.subcore_barrier()
    _edge_sweep(u_sh, acc, src_v, dst_v, rows_v, gsem, ssem)
    plsc.subcore_barrier()
    a0 = a_v[pl.ds(0, 16)]
    a1 = a_v[pl.ds(16, 16)]
    b0 = b_v[pl.ds(0, 16)]
    b1 = b_v[pl.ds(16, 16)]

    def slab(t, carry):
        base = s * STRIPE + t * SLAB
        pltpu.sync_copy(acc.at[pl.ds(base, SLAB)], sbuf)

        def row(j, carry2):
            d = plsc.load_gather(dis_v,
                                 [jnp.full((16,), t * SLAB + j, jnp.int32)])
            r0 = sbuf[j, pl.ds(0, 16)]
            r1 = sbuf[j, pl.ds(16, 16)]
            sbuf[j, pl.ds(0, 16)] = d * jnp.maximum(d * r0 * a0 + b0, 0.0)
            sbuf[j, pl.ds(16, 16)] = d * jnp.maximum(d * r1 * a1 + b1, 0.0)
            return carry2

        lax.fori_loop(0, SLAB, row, 0, unroll=4)
        pltpu.sync_copy(sbuf, u_sh.at[pl.ds(base, SLAB)])
        pltpu.sync_copy(sbuf, acc.at[pl.ds(base, SLAB)])
        return carry

    lax.fori_loop(0, STRIPE // SLAB, slab, 0)
    plsc.subcore_barrier()
    _edge_sweep(u_sh, acc, src_v, dst_v, rows_v, gsem, ssem)
    plsc.subcore_barrier()

    @pl.when(c == 0)
    def _():
        pltpu.sync_copy(acc.at[pl.ds(s * STRIPE, STRIPE)],
                        olo_hbm.at[pl.ds(s * STRIPE, STRIPE)])

    @pl.when(c == 1)
    def _():
        pltpu.sync_copy(acc.at[pl.ds(s * STRIPE, STRIPE)],
                        ohi_hbm.at[pl.ds(s * STRIPE, STRIPE)])


@functools.cache
def _prop23_call():
    return pl.kernel(
        _prop23_body,
        out_type=[jax.ShapeDtypeStruct((NP, HH), jnp.float32),
                  jax.ShapeDtypeStruct((NP, HH), jnp.float32)],
        mesh=_mesh(),
        scratch_types=[
            pltpu.VMEM((PCHUNKS, C), jnp.int32),
            pltpu.VMEM((PCHUNKS, C), jnp.int32),
            pltpu.VMEM((GROUP, C, HH), jnp.float32),
            pltpu.VMEM((SLAB, HH), jnp.float32),
            pltpu.VMEM((STRIPE,), jnp.float32),
            pltpu.VMEM((HH,), jnp.float32),
            pltpu.VMEM((HH,), jnp.float32),
            pltpu.VMEM_SHARED((NP, HH), jnp.float32),
            pltpu.VMEM_SHARED((NP, HH), jnp.float32),
            pltpu.SemaphoreType.DMA,
            pltpu.SemaphoreType.DMA,
        ],
        compiler_params=pltpu.CompilerParams(use_tc_tiling_on_sc=False,
                                             needs_layout_passes=False),
    )


def _deg_body(dsts_hbm, ones_hbm, zeros_hbm, out_hbm,
              dst_v, ones_v, acc, ssem):
    c = lax.axis_index("c")
    s = lax.axis_index("s")
    w = s * 2 + c
    pltpu.sync_copy(zeros_hbm.at[pl.ds(s * STRIPE, STRIPE)],
                    acc.at[pl.ds(s * STRIPE, STRIPE)])
    pltpu.sync_copy(ones_hbm, ones_v)
    pltpu.sync_copy(dsts_hbm.at[pl.ds(w * DCHUNKS, DCHUNKS)], dst_v)
    plsc.subcore_barrier()

    def group(g, carry):
        @pl.when(g > 0)
        def _():
            for b in range(DEG_GROUP):
                pltpu.make_async_copy(ones_v, acc.at[dst_v.at[0]], ssem).wait()
        for b in range(DEG_GROUP):
            pltpu.async_copy(ones_v, acc.at[dst_v.at[g * DEG_GROUP + b]], ssem,
                             add=True)
        return carry

    lax.fori_loop(0, DCHUNKS // DEG_GROUP, group, 0)
    for b in range(DEG_GROUP):
        pltpu.make_async_copy(ones_v, acc.at[dst_v.at[0]], ssem).wait()
    plsc.subcore_barrier()
    pltpu.sync_copy(acc.at[pl.ds(s * STRIPE, STRIPE)],
                    out_hbm.at[c, pl.ds(s * STRIPE, STRIPE)])


@functools.cache
def _deg_call():
    return pl.kernel(
        _deg_body,
        out_type=jax.ShapeDtypeStruct((2, NP, DEG_W), jnp.float32),
        mesh=_mesh(),
        scratch_types=[
            pltpu.VMEM((DCHUNKS, C), jnp.int32),
            pltpu.VMEM((C, DEG_W), jnp.float32),
            pltpu.VMEM_SHARED((NP, DEG_W), jnp.float32),
            pltpu.SemaphoreType.DMA,
        ],
        compiler_params=pltpu.CompilerParams(use_tc_tiling_on_sc=False),
    )


_RB = 1024
_GRID = NP // _RB

_halves = pl.BlockSpec((2, _RB, HH), lambda i: (0, i, 0))


def _split(u, out_ref):
    out_ref[0] = u[:, :HH]
    out_ref[1] = u[:, HH:]


def _tc1a_body(x_ref, w1_ref, h_ref):
    h_ref[...] = jnp.dot(x_ref[...], w1_ref[...],
                         preferred_element_type=jnp.float32)


def _tc1a(x, w1):
    return pl.pallas_call(
        _tc1a_body,
        grid=(_GRID,),
        in_specs=[
            pl.BlockSpec((_RB, 128), lambda i: (i, 0)),
            pl.BlockSpec((128, DH), lambda i: (0, 0)),
        ],
        out_specs=pl.BlockSpec((_RB, DH), lambda i: (i, 0)),
        out_shape=jax.ShapeDtypeStruct((NP, DH), jnp.float32),
    )(x, w1)


def _tc1b_body(h_ref, d0_ref, d1_ref, u1_ref, dis_ref):
    cnt = d0_ref[:, 0:1] + d1_ref[:, 0:1]
    dis = lax.rsqrt(cnt + 1.0)
    dis_ref[...] = dis
    _split(dis * h_ref[...], u1_ref)


def _tc1b(h, d0, d1):
    return pl.pallas_call(
        _tc1b_body,
        grid=(_GRID,),
        in_specs=[
            pl.BlockSpec((_RB, DH), lambda i: (i, 0)),
            pl.BlockSpec((_RB, DEG_W), lambda i: (i, 0)),
            pl.BlockSpec((_RB, DEG_W), lambda i: (i, 0)),
        ],
        out_specs=[
            _halves,
            pl.BlockSpec((_RB, 1), lambda i: (i, 0)),
        ],
        out_shape=[
            jax.ShapeDtypeStruct((2, NP, HH), jnp.float32),
            jax.ShapeDtypeStruct((NP, 1), jnp.float32),
        ],
    )(h, d0, d1)


def _tc2_body(s_ref, dis_ref, b_ref, g_ref, be_ref, rm_ref,
              rv_ref, w_ref, out_ref):
    dis = dis_ref[...]
    su = jnp.concatenate([s_ref[0], s_ref[1]], axis=1)
    t = dis * su + b_ref[...]
    t = (t - rm_ref[...]) * lax.rsqrt(rv_ref[...] + 1e-5) * g_ref[...] + be_ref[...]
    t = jnp.maximum(t, 0.0)
    _split(dis * jnp.dot(t, w_ref[...], preferred_element_type=jnp.float32),
           out_ref)


def _tc2(s, dis, b, g, be, rm, rv, w):
    vec = pl.BlockSpec((1, DH), lambda i: (0, 0))
    return pl.pallas_call(
        _tc2_body,
        grid=(_GRID,),
        in_specs=[_halves, pl.BlockSpec((_RB, 1), lambda i: (i, 0)),
                  vec, vec, vec, vec, vec,
                  pl.BlockSpec((DH, DH), lambda i: (0, 0))],
        out_specs=_halves,
        out_shape=jax.ShapeDtypeStruct((2, NP, HH), jnp.float32),
    )(s, dis, b, g, be, rm, rv, w)


def _tc3_body(s_ref, dis_ref, b_ref, g_ref, be_ref, rm_ref, rv_ref, out_ref):
    dis = dis_ref[...]
    su = jnp.concatenate([s_ref[0], s_ref[1]], axis=1)
    t = dis * su + b_ref[...]
    t = (t - rm_ref[...]) * lax.rsqrt(rv_ref[...] + 1e-5) * g_ref[...] + be_ref[...]
    t = jnp.maximum(t, 0.0)
    _split(dis * t, out_ref)


def _tc3(s, dis, b, g, be, rm, rv):
    vec = pl.BlockSpec((1, DH), lambda i: (0, 0))
    return pl.pallas_call(
        _tc3_body,
        grid=(_GRID,),
        in_specs=[_halves, pl.BlockSpec((_RB, 1), lambda i: (i, 0)),
                  vec, vec, vec, vec, vec],
        out_specs=_halves,
        out_shape=jax.ShapeDtypeStruct((2, NP, HH), jnp.float32),
    )(s, dis, b, g, be, rm, rv)


def _tc4_body(s_ref, dis_ref, w_ref, b_ref, out_ref):
    p = dis_ref[...] * jnp.concatenate([s_ref[0], s_ref[1]], axis=1)
    out_ref[...] = jnp.dot(p, w_ref[...],
                           preferred_element_type=jnp.float32) + b_ref[...]


def _tc4(s, dis, w, b):
    return pl.pallas_call(
        _tc4_body,
        grid=(_GRID,),
        in_specs=[_halves, pl.BlockSpec((_RB, 1), lambda i: (i, 0)),
                  pl.BlockSpec((DH, 128), lambda i: (0, 0)),
                  pl.BlockSpec((1, 128), lambda i: (0, 0))],
        out_specs=pl.BlockSpec((_RB, 128), lambda i: (i, 0)),
        out_shape=jax.ShapeDtypeStruct((NP, 128), jnp.float32),
    )(s, dis, w, b)


def kernel(x, edge_index, W1, b1, g1, be1, rm1, rv1, W2, b2, g2, be2, rm2,
           rv2, W3, b3):
    src = edge_index[0].astype(jnp.int32)
    dst = edge_index[1].astype(jnp.int32)
    pad = E_PAD - src.shape[0]
    srcs = jnp.concatenate([src, jnp.zeros((pad,), jnp.int32)]).reshape(-1, C)
    dsts = jnp.concatenate([dst, jnp.full((pad,), DUMMY, jnp.int32)]).reshape(-1, C)
    ones = jnp.ones((C, DEG_W), jnp.float32)
    zeros = jnp.zeros((NP, DEG_W), jnp.float32)
    xp = jnp.pad(x, ((0, NP - x.shape[0]), (0, 0)))

    h1 = _tc1a(xp, W1)
    degp = _deg_call()(dsts, ones, zeros)
    u1, dis = _tc1b(h1, degp[0], degp[1])
    s1 = _prop_call()(u1, srcs, dsts)
    u2 = _tc2(s1, dis, b1.reshape(1, DH), g1.reshape(1, DH),
              be1.reshape(1, DH), rm1.reshape(1, DH), rv1.reshape(1, DH), W2)
    a2 = g2 * lax.rsqrt(rv2 + 1e-5)
    b2f = (b2 - rm2) * a2 + be2
    s3lo, s3hi = _prop23_call()(u2[0], u2[1], srcs, dsts, dis.reshape(NP),
                                a2.reshape(2, HH), b2f.reshape(2, HH))
    s3 = jnp.stack([s3lo, s3hi])
    w3p = jnp.pad(W3, ((0, 0), (0, 128 - W3.shape[1])))
    b3p = jnp.pad(b3, (0, 128 - b3.shape[0])).reshape(1, 128)
    outp = _tc4(s3, dis, w3p, b3p)
    return outp[:N, :b3.shape[0]]
